# baseline (speedup 1.0000x reference)
import functools

import jax
import jax.numpy as jnp
from jax import lax
from jax.experimental import pallas as pl
from jax.experimental.pallas import tpu as pltpu
from jax.experimental.pallas import tpu_sc as plsc

SEQ = 200
BATCH = 4096
D = 64
VOCAB = 1000000
NW = 32

CB1 = 128
NBLK = (VOCAB + CB1 - 1) // CB1

BPW = BATCH // NW
NREG = D // 16
KH = 4
NGRP = SEQ // (2 * KH)
UB = 4

_mesh = plsc.VectorSubcoreMesh(core_axis_name="c", subcore_axis_name="s")


def _tc_transpose_body(tabt_ref, out_ref):
    x = tabt_ref[...]
    q = lax.broadcasted_iota(jnp.int32, (D, CB1), 0)
    c = lax.broadcasted_iota(jnp.int32, (D, CB1), 1)
    sel_lo = (c == 2 * q).astype(jnp.float32)
    sel_hi = (c == 2 * q + 1).astype(jnp.float32)
    dn = (((1,), (1,)), ((), ()))
    out_ref[:, 0:D] = lax.dot_general(
        sel_lo, x, dn, preferred_element_type=jnp.float32)
    out_ref[:, D:CB1] = lax.dot_general(
        sel_hi, x, dn, preferred_element_type=jnp.float32)


_tc_transpose = pl.pallas_call(
    _tc_transpose_body,
    grid=(NBLK,),
    in_specs=[pl.BlockSpec((D, CB1), lambda j: (0, j))],
    out_specs=pl.BlockSpec((D, CB1), lambda j: (j, 0)),
    out_shape=jax.ShapeDtypeStruct((VOCAB // 2, CB1), jnp.float32),
)


@functools.partial(
    pl.kernel,
    mesh=_mesh,
    out_type=jax.ShapeDtypeStruct((BATCH, D), jnp.float32),
    compiler_params=pltpu.CompilerParams(use_tc_tiling_on_sc=False),
    scratch_types=[
        pltpu.VMEM((SEQ, 2, BPW), jnp.int32),
        pltpu.VMEM((2, KH, 2 * BPW, 32), jnp.float32),
        pltpu.VMEM((BPW, D), jnp.float32),
        pltpu.SemaphoreType.DMA,
        pltpu.SemaphoreType.DMA,
    ],
)
def _sum_embed(idx_hbm, tab2_hbm, out_hbm, idx_v, ring_v, acc_v, sem_a, sem_b):
    wid = lax.axis_index("s") * 2 + lax.axis_index("c")
    pltpu.sync_copy(idx_hbm.at[:, wid], idx_v)

    def fire(s0, half, sem):
        for j in range(KH):
            for hh in range(2):
                pltpu.async_copy(
                    tab2_hbm.at[idx_v.at[s0 + j, hh]],
                    ring_v.at[half, j, pl.ds(hh * BPW, BPW)],
                    sem)

    def drain(half, sem):
        for j in range(KH):
            for hh in range(2):
                pltpu.make_async_copy(
                    tab2_hbm.at[idx_v.at[0, 0]],
                    ring_v.at[half, j, pl.ds(hh * BPW, BPW)],
                    sem).wait()

    def reduce_half(half):
        for j in range(KH):
            def red(bi, c, j=j):
                for u in range(UB):
                    b = bi * UB + u
                    for k in range(NREG):
                        plsc.addupdate(
                            acc_v.at[b, pl.ds(k * 16, 16)],
                            ring_v[half, j, 2 * b + k // 2,
                                   pl.ds((k % 2) * 16, 16)],
                        )
                return c
            lax.fori_loop(0, BPW // UB, red, 0)

    zvec = jnp.zeros((16,), jnp.float32)

    def zero(bi, c):
        for u in range(UB):
            for k in range(NREG):
                acc_v[bi * UB + u, pl.ds(k * 16, 16)] = zvec
        return c

    lax.fori_loop(0, BPW // UB, zero, 0)

    fire(0, 0, sem_a)

    def grp(g, carry):
        s0 = g * 2 * KH
        fire(s0 + KH, 1, sem_b)
        drain(0, sem_a)
        reduce_half(0)

        @pl.when(g < NGRP - 1)
        def _():
            fire(s0 + 2 * KH, 0, sem_a)

        drain(1, sem_b)
        reduce_half(1)
        return carry

    lax.fori_loop(0, NGRP, grp, 0)
    pltpu.sync_copy(acc_v, out_hbm.at[pl.ds(wid * BPW, BPW)])


def kernel(indices, table):
    pairs = jnp.stack((indices * 2, indices * 2 + 1), axis=-1)
    idx4 = pairs.reshape(SEQ, NW, 2, BPW)
    tab2 = _tc_transpose(table.T).reshape(2 * VOCAB, 32)
    return _sum_embed(idx4, tab2)


# speedup vs baseline: 6.1071x; 6.1071x over previous
import functools

import jax
import jax.numpy as jnp
from jax import lax
from jax.experimental import pallas as pl
from jax.experimental.pallas import tpu as pltpu
from jax.experimental.pallas import tpu_sc as plsc

SEQ = 200
BATCH = 4096
D = 64
NW = 32
BPW = BATCH // NW
NREG = D // 16
KH = 4
NGRP = SEQ // (2 * KH)
UB = 4

_mesh = plsc.VectorSubcoreMesh(core_axis_name="c", subcore_axis_name="s")


@functools.partial(
    pl.kernel,
    mesh=_mesh,
    out_type=jax.ShapeDtypeStruct((BATCH, D), jnp.float32),
    compiler_params=pltpu.CompilerParams(use_tc_tiling_on_sc=False),
    scratch_types=[
        pltpu.VMEM((SEQ, BPW), jnp.int32),
        pltpu.VMEM((2, KH, BPW, D), jnp.float32),
        pltpu.VMEM((BPW, D), jnp.float32),
        pltpu.SemaphoreType.DMA,
        pltpu.SemaphoreType.DMA,
    ],
)
def _sum_embed(idx_hbm, tab_hbm, out_hbm, idx_v, ring_v, acc_v, sem_a, sem_b):
    wid = lax.axis_index("s") * 2 + lax.axis_index("c")
    pltpu.sync_copy(idx_hbm.at[:, pl.ds(wid * BPW, BPW)], idx_v)

    def fire(s0, half, sem):
        for j in range(KH):
            pltpu.async_copy(
                tab_hbm.at[idx_v.at[s0 + j]], ring_v.at[half, j], sem)

    def drain(s0, half, sem):
        for j in range(KH):
            pltpu.make_async_copy(
                tab_hbm.at[idx_v.at[s0 + j]], ring_v.at[half, j], sem).wait()

    def reduce_half(half):
        for j in range(KH):
            def red(bi, c, j=j):
                for u in range(UB):
                    b = bi * UB + u
                    for k in range(NREG):
                        plsc.addupdate(
                            acc_v.at[b, pl.ds(k * 16, 16)],
                            ring_v[half, j, b, pl.ds(k * 16, 16)],
                        )
                return c
            lax.fori_loop(0, BPW // UB, red, 0)

    zvec = jnp.zeros((16,), jnp.float32)

    def zero(bi, c):
        for u in range(UB):
            for k in range(NREG):
                acc_v[bi * UB + u, pl.ds(k * 16, 16)] = zvec
        return c

    lax.fori_loop(0, BPW // UB, zero, 0)

    fire(0, 0, sem_a)

    def grp(g, carry):
        s0 = g * 2 * KH
        fire(s0 + KH, 1, sem_b)
        drain(s0, 0, sem_a)
        reduce_half(0)

        @pl.when(g < NGRP - 1)
        def _():
            fire(s0 + 2 * KH, 0, sem_a)

        drain(s0 + KH, 1, sem_b)
        reduce_half(1)
        return carry

    lax.fori_loop(0, NGRP, grp, 0)
    pltpu.sync_copy(acc_v, out_hbm.at[pl.ds(wid * BPW, BPW)])


def kernel(indices, table):
    return _sum_embed(indices, table)


# speedup vs baseline: 7.3792x; 1.2083x over previous
import functools

import jax
import jax.numpy as jnp
from jax import lax
from jax.experimental import pallas as pl
from jax.experimental.pallas import tpu as pltpu
from jax.experimental.pallas import tpu_sc as plsc

SEQ = 200
BATCH = 4096
D = 64
VOCAB = 1000000
NW = 32

SUB = 128
NSUB = 16
CB1 = SUB * NSUB
NBLK = (VOCAB + CB1 - 1) // CB1

BPW = BATCH // NW
NREG = D // 16
KH = 4
NGRP = SEQ // (2 * KH)
UB = 4

_mesh = plsc.VectorSubcoreMesh(core_axis_name="c", subcore_axis_name="s")


def _tc_transpose_body(tabt_ref, out_ref):
    q = lax.broadcasted_iota(jnp.int32, (D, SUB), 0)
    c = lax.broadcasted_iota(jnp.int32, (D, SUB), 1)
    sel_lo = (c == 2 * q).astype(jnp.float32)
    sel_hi = (c == 2 * q + 1).astype(jnp.float32)
    dn = (((1,), (1,)), ((), ()))
    for m in range(NSUB):
        xm = tabt_ref[:, m * SUB:(m + 1) * SUB]
        out_ref[m * D:(m + 1) * D, 0:D] = lax.dot_general(
            sel_lo, xm, dn, preferred_element_type=jnp.float32)
        out_ref[m * D:(m + 1) * D, D:2 * D] = lax.dot_general(
            sel_hi, xm, dn, preferred_element_type=jnp.float32)


_tc_transpose = pl.pallas_call(
    _tc_transpose_body,
    grid=(NBLK,),
    in_specs=[pl.BlockSpec((D, CB1), lambda j: (0, j))],
    out_specs=pl.BlockSpec((CB1 // 2, 2 * D), lambda j: (j, 0)),
    out_shape=jax.ShapeDtypeStruct((VOCAB // 2, 2 * D), jnp.float32),
)


@functools.partial(
    pl.kernel,
    mesh=_mesh,
    out_type=jax.ShapeDtypeStruct((BATCH, D), jnp.float32),
    compiler_params=pltpu.CompilerParams(use_tc_tiling_on_sc=False),
    scratch_types=[
        pltpu.VMEM((SEQ, 2, BPW), jnp.int32),
        pltpu.VMEM((2, KH, 2 * BPW, 32), jnp.float32),
        pltpu.VMEM((BPW, D), jnp.float32),
        pltpu.SemaphoreType.DMA,
        pltpu.SemaphoreType.DMA,
    ],
)
def _sum_embed(idx_hbm, tab2_hbm, out_hbm, idx_v, ring_v, acc_v, sem_a, sem_b):
    wid = lax.axis_index("s") * 2 + lax.axis_index("c")
    pltpu.sync_copy(idx_hbm.at[:, wid], idx_v)

    def fire(s0, half, sem):
        for j in range(KH):
            for hh in range(2):
                pltpu.async_copy(
                    tab2_hbm.at[idx_v.at[s0 + j, hh]],
                    ring_v.at[half, j, pl.ds(hh * BPW, BPW)],
                    sem)

    def drain(half, sem):
        for j in range(KH):
            for hh in range(2):
                pltpu.make_async_copy(
                    tab2_hbm.at[idx_v.at[0, 0]],
                    ring_v.at[half, j, pl.ds(hh * BPW, BPW)],
                    sem).wait()

    def reduce_half(half):
        for j in range(KH):
            def red(bi, c, j=j):
                for u in range(UB):
                    b = bi * UB + u
                    for k in range(NREG):
                        plsc.addupdate(
                            acc_v.at[b, pl.ds(k * 16, 16)],
                            ring_v[half, j, 2 * b + k // 2,
                                   pl.ds((k % 2) * 16, 16)],
                        )
                return c
            lax.fori_loop(0, BPW // UB, red, 0)

    zvec = jnp.zeros((16,), jnp.float32)

    def zero(bi, c):
        for u in range(UB):
            for k in range(NREG):
                acc_v[bi * UB + u, pl.ds(k * 16, 16)] = zvec
        return c

    lax.fori_loop(0, BPW // UB, zero, 0)

    fire(0, 0, sem_a)

    def grp(g, carry):
        s0 = g * 2 * KH
        fire(s0 + KH, 1, sem_b)
        drain(0, sem_a)
        reduce_half(0)

        @pl.when(g < NGRP - 1)
        def _():
            fire(s0 + 2 * KH, 0, sem_a)

        drain(1, sem_b)
        reduce_half(1)
        return carry

    lax.fori_loop(0, NGRP, grp, 0)
    pltpu.sync_copy(acc_v, out_hbm.at[pl.ds(wid * BPW, BPW)])


def kernel(indices, table):
    pairs = jnp.stack((indices * 2, indices * 2 + 1), axis=-1)
    idx4 = pairs.reshape(SEQ, NW, 2, BPW)
    tab2 = _tc_transpose(table.T).reshape(2 * VOCAB, 32)
    return _sum_embed(idx4, tab2)
